# baseline (speedup 1.0000x reference)
import functools

import jax
import jax.numpy as jnp
from jax import lax
from jax.experimental import pallas as pl
from jax.experimental.pallas import tpu as pltpu
from jax.experimental.pallas import tpu_sc as plsc

B = 1024
S = 200
D = 128
K = 50
KT = 56
KP = 64

NC = 2
NS = 16
NW = NC * NS
BPW = B // NW
NBUF = 4
L = 16

_mesh = plsc.VectorSubcoreMesh(core_axis_name="c", subcore_axis_name="s")


@functools.partial(
    pl.kernel,
    mesh=_mesh,
    out_type=jax.ShapeDtypeStruct((B, KT, D), jnp.float32),
    scratch_types=[
        pltpu.VMEM((KP,), jnp.int32),
        pltpu.VMEM((BPW, KP), jnp.int32),
        pltpu.VMEM((NBUF, KT, D), jnp.float32),
        pltpu.SemaphoreType.DMA,
        pltpu.SemaphoreType.DMA,
    ],
)
def _gather(x_hbm, idx_hbm, out_hbm, idx_v, gidx, buf, gsem, wsem):
    wid = lax.axis_index("s") * NC + lax.axis_index("c")
    base_batch = wid * BPW

    pltpu.sync_copy(idx_hbm, idx_v)

    for v in range(KP // L):
        iv = idx_v[pl.ds(v * L, L)]
        for i in range(BPW):
            off = jnp.full((L,), (base_batch + i) * S, jnp.int32)
            gidx[i, pl.ds(v * L, L)] = iv + off

    def gstart(i):
        return pltpu.async_copy(
            x_hbm.at[gidx.at[i, pl.ds(0, KT)]], buf.at[i % NBUF], gsem)

    gh = [None] * BPW
    wh = [None] * BPW
    for i in range(NBUF):
        gh[i] = gstart(i)
    for i in range(BPW):
        gh[i].wait()
        wh[i] = pltpu.async_copy(
            buf.at[i % NBUF], out_hbm.at[base_batch + i], wsem)
        ni = i + NBUF
        if ni < BPW:
            wh[i].wait()
            gh[ni] = gstart(ni)
    for i in range(BPW - NBUF, BPW):
        wh[i].wait()


def kernel(x, index):
    x2d = x.reshape(B * S, D)
    idx_pad = jnp.zeros((KP,), jnp.int32).at[:K].set(index)
    padded = _gather(x2d, idx_pad)
    return padded[:, :K, :]


# speedup vs baseline: 1.1967x; 1.1967x over previous
import functools

import jax
import jax.numpy as jnp
from jax import lax
from jax.experimental import pallas as pl
from jax.experimental.pallas import tpu as pltpu
from jax.experimental.pallas import tpu_sc as plsc

B = 1024
S = 200
D = 128
K = 50
KP = 64

NC = 2
NS = 16
NW = NC * NS
BPW = B // NW
NBUF = 8
L = 16

_mesh = plsc.VectorSubcoreMesh(core_axis_name="c", subcore_axis_name="s")


@functools.partial(
    pl.kernel,
    mesh=_mesh,
    out_type=jax.ShapeDtypeStruct((B, K, D), jnp.float32),
    scratch_types=[
        pltpu.VMEM((KP,), jnp.int32),
        pltpu.VMEM((BPW, KP), jnp.int32),
        pltpu.VMEM((NBUF, K, D), jnp.float32),
        pltpu.SemaphoreType.DMA,
        pltpu.SemaphoreType.DMA,
    ],
)
def _gather(x_hbm, idx_hbm, out_hbm, idx_v, gidx, buf, gsem, wsem):
    wid = lax.axis_index("s") * NC + lax.axis_index("c")
    base_batch = wid * BPW

    pltpu.sync_copy(idx_hbm, idx_v)

    for v in range(KP // L):
        iv = idx_v[pl.ds(v * L, L)]
        for i in range(BPW):
            off = jnp.full((L,), (base_batch + i) * S, jnp.int32)
            gidx[i, pl.ds(v * L, L)] = iv + off

    def gstart(i):
        return pltpu.async_copy(
            x_hbm.at[gidx.at[i, pl.ds(0, K)]], buf.at[i % NBUF], gsem)

    gh = [None] * BPW
    wh = [None] * BPW
    for i in range(NBUF):
        gh[i] = gstart(i)
    for i in range(BPW):
        gh[i].wait()
        wh[i] = pltpu.async_copy(
            buf.at[i % NBUF], out_hbm.at[base_batch + i], wsem)
        ni = i + NBUF
        if ni < BPW:
            wh[i].wait()
            gh[ni] = gstart(ni)
    for i in range(BPW - NBUF, BPW):
        wh[i].wait()


def kernel(x, index):
    x2d = x.reshape(B * S, D)
    idx_pad = jnp.zeros((KP,), jnp.int32).at[:K].set(index)
    return _gather(x2d, idx_pad)
